# baseline (speedup 1.0000x reference)
import functools

import jax
import jax.numpy as jnp
from jax import lax
from jax.experimental import pallas as pl
from jax.experimental.pallas import tpu as pltpu
from jax.experimental.pallas import tpu_sc as plsc

_K = 32
_B = 16384
_NC = 2
_NS = 16
_NW = _NC * _NS
_BPW = _B // _NW
_L = 16
_R = 4
_W = 128
_CH = 256
_NCH = _BPW // _CH


def _body(users_hbm, items_hbm, neg_hbm, p_hbm, q_hbm, pos_out, neg_out,
          idx_u, idx_i, idx_n, row_u, row_i, row_n,
          rows_u, rows_i, rows_n, pos_v, neg_v, sem):
    wid = lax.axis_index("s") * _NC + lax.axis_index("c")
    base = wid * _BPW

    pltpu.sync_copy(users_hbm.at[pl.ds(base, _BPW)], idx_u)
    pltpu.sync_copy(items_hbm.at[pl.ds(base, _BPW)], idx_i)
    pltpu.sync_copy(neg_hbm.at[pl.ds(base, _BPW)], idx_n)

    def rowify(j, carry):
        s = pl.ds(j * _L, _L)
        row_u[s] = lax.shift_right_logical(idx_u[s], 2)
        row_i[s] = lax.shift_right_logical(idx_i[s], 2)
        row_n[s] = lax.shift_right_logical(idx_n[s], 2)
        return carry

    lax.fori_loop(0, _BPW // _L, rowify, 0)

    iota = lax.iota(jnp.int32, _L)
    zeros = jnp.zeros((_L,), jnp.float32)
    three = jnp.full((_L,), 3, jnp.int32)

    def chunk(c, carry):
        cs = pl.ds(c * _CH, _CH)
        cu = pltpu.async_copy(p_hbm.at[row_u.at[cs]], rows_u, sem)
        ci = pltpu.async_copy(q_hbm.at[row_i.at[cs]], rows_i, sem)
        cn = pltpu.async_copy(q_hbm.at[row_n.at[cs]], rows_n, sem)
        cu.wait()
        ci.wait()
        cn.wait()

        def group(g, carry2):
            m = g * _L + iota
            gs = pl.ds(c * _CH + g * _L, _L)
            su = lax.mul(jnp.bitwise_and(idx_u[gs], three), jnp.full((_L,), _K, jnp.int32))
            si = lax.mul(jnp.bitwise_and(idx_i[gs], three), jnp.full((_L,), _K, jnp.int32))
            sn = lax.mul(jnp.bitwise_and(idx_n[gs], three), jnp.full((_L,), _K, jnp.int32))
            acc_p = zeros
            acc_n = zeros
            for k in range(_K):
                kk = jnp.full((_L,), k, jnp.int32)
                u = plsc.load_gather(rows_u, [m, su + kk])
                qi = plsc.load_gather(rows_i, [m, si + kk])
                qn = plsc.load_gather(rows_n, [m, sn + kk])
                acc_p = acc_p + u * qi
                acc_n = acc_n + u * qn
            pos_v[gs] = acc_p
            neg_v[gs] = acc_n
            return carry2

        lax.fori_loop(0, _CH // _L, group, 0)
        return carry

    lax.fori_loop(0, _NCH, chunk, 0)

    pltpu.sync_copy(pos_v, pos_out.at[pl.ds(base, _BPW)])
    pltpu.sync_copy(neg_v, neg_out.at[pl.ds(base, _BPW)])


@jax.jit
def _run(users, items, neg_items, p, q):
    p4 = p.reshape(p.shape[0] // _R, _W)
    q4 = q.reshape(q.shape[0] // _R, _W)
    mesh = plsc.VectorSubcoreMesh(core_axis_name="c", subcore_axis_name="s")
    f = pl.kernel(
        _body,
        mesh=mesh,
        out_type=(
            jax.ShapeDtypeStruct((_B,), jnp.float32),
            jax.ShapeDtypeStruct((_B,), jnp.float32),
        ),
        scratch_types=[
            pltpu.VMEM((_BPW,), jnp.int32),
            pltpu.VMEM((_BPW,), jnp.int32),
            pltpu.VMEM((_BPW,), jnp.int32),
            pltpu.VMEM((_BPW,), jnp.int32),
            pltpu.VMEM((_BPW,), jnp.int32),
            pltpu.VMEM((_BPW,), jnp.int32),
            pltpu.VMEM((_CH, _W), jnp.float32),
            pltpu.VMEM((_CH, _W), jnp.float32),
            pltpu.VMEM((_CH, _W), jnp.float32),
            pltpu.VMEM((_BPW,), jnp.float32),
            pltpu.VMEM((_BPW,), jnp.float32),
            pltpu.SemaphoreType.DMA,
        ],
        compiler_params=pltpu.CompilerParams(
            needs_layout_passes=False, use_tc_tiling_on_sc=True
        ),
    )
    return f(users, items, neg_items, p4, q4)


def kernel(users, items, neg_items, P, Q):
    users = users.astype(jnp.int32)
    items = items.astype(jnp.int32)
    neg_items = neg_items.astype(jnp.int32)
    return _run(users, items, neg_items, P, Q)
